# baseline (speedup 1.0000x reference)
import functools

import jax
import jax.numpy as jnp
from jax import lax
from jax.experimental import pallas as pl
from jax.experimental.pallas import tpu as pltpu
from jax.experimental.pallas import tpu_sc as plsc

SEQ = 8192
B = 4
D = 1024
NC = 2
NS = 16
NW = NC * NS
ROWS_PW = SEQ // NW
R = 8
NCH = ROWS_PW // R
NB = 2


def _sc_body(x_hbm, pe_hbm, o_hbm, xb, peb, rx, rp, ws):
    wid = lax.axis_index("s") * NC + lax.axis_index("c")
    base = wid * ROWS_PW

    def x_copy(i, slot):
        return pltpu.make_async_copy(
            x_hbm.at[pl.ds(base + i * R, R)], xb.at[slot], rx.at[slot])

    def pe_copy(i, slot):
        return pltpu.make_async_copy(
            pe_hbm.at[pl.ds(base + i * R, R)], peb.at[slot], rp.at[slot])

    def o_copy(i, slot):
        return pltpu.make_async_copy(
            xb.at[slot], o_hbm.at[pl.ds(base + i * R, R)], ws.at[slot])

    x_copy(0, 0).start()
    pe_copy(0, 0).start()

    def step(i, carry):
        slot = lax.rem(i, NB)
        nslot = lax.rem(i + 1, NB)

        @pl.when(i + 1 < NCH)
        def _():
            @pl.when(i + 1 >= NB)
            def _():
                o_copy(i + 1 - NB, nslot).wait()
            x_copy(i + 1, nslot).start()
            pe_copy(i + 1, nslot).start()

        x_copy(i, slot).wait()
        pe_copy(i, slot).wait()

        def row(r, carry2):
            for j in range(D // 16):
                pe_v = peb[slot, r, pl.ds(j * 16, 16)]
                for b in range(B):
                    xb[slot, r, b, pl.ds(j * 16, 16)] = (
                        xb[slot, r, b, pl.ds(j * 16, 16)] + pe_v)
            return carry2

        lax.fori_loop(0, R, row, 0)
        o_copy(i, slot).start()
        return carry

    lax.fori_loop(0, NCH, step, 0)

    for k in range(NB):
        i = NCH - NB + k
        o_copy(i, lax.rem(jnp.int32(i), NB)).wait()


def kernel(x, pos_emb):
    seq_len, batch, d_model = x.shape
    sc = pl.kernel(
        _sc_body,
        out_type=jax.ShapeDtypeStruct((seq_len, batch, d_model), x.dtype),
        mesh=plsc.VectorSubcoreMesh(core_axis_name="c", subcore_axis_name="s"),
        scratch_types=[
            pltpu.VMEM((NB, R, B, D), x.dtype),
            pltpu.VMEM((NB, R, D), x.dtype),
            pltpu.SemaphoreType.DMA((NB,)),
            pltpu.SemaphoreType.DMA((NB,)),
            pltpu.SemaphoreType.DMA((NB,)),
        ],
    )
    return sc(x, pos_emb)


# speedup vs baseline: 1.2150x; 1.2150x over previous
import functools

import jax
import jax.numpy as jnp
from jax import lax
from jax.experimental import pallas as pl
from jax.experimental.pallas import tpu as pltpu
from jax.experimental.pallas import tpu_sc as plsc

SEQ = 8192
B = 4
D = 1024

S_SC = 2560
S_TC = SEQ - S_SC
S_BLK = 512

NC = 2
NS = 16
NW = NC * NS
ROWS_PW = S_SC // NW
R = 8
NCH = ROWS_PW // R
NB = 2


def _tc_body(x_ref, pe_ref, o_ref):
    pe = pe_ref[...]
    o_ref[...] = x_ref[...] + pe[:, None, :]


def _sc_body(x_hbm, pe_hbm, o_hbm, xb, peb, rx, rp, ws):
    wid = lax.axis_index("s") * NC + lax.axis_index("c")
    src = S_TC + wid * ROWS_PW
    dst = wid * ROWS_PW

    def x_copy(i, slot):
        return pltpu.make_async_copy(
            x_hbm.at[pl.ds(src + i * R, R)], xb.at[slot], rx.at[slot])

    def pe_copy(i, slot):
        return pltpu.make_async_copy(
            pe_hbm.at[pl.ds(src + i * R, R)], peb.at[slot], rp.at[slot])

    def o_copy(i, slot):
        return pltpu.make_async_copy(
            xb.at[slot], o_hbm.at[pl.ds(dst + i * R, R)], ws.at[slot])

    x_copy(0, 0).start()
    pe_copy(0, 0).start()

    def step(i, carry):
        slot = lax.rem(i, NB)
        nslot = lax.rem(i + 1, NB)

        @pl.when(i + 1 < NCH)
        def _():
            @pl.when(i + 1 >= NB)
            def _():
                o_copy(i + 1 - NB, nslot).wait()
            x_copy(i + 1, nslot).start()
            pe_copy(i + 1, nslot).start()

        x_copy(i, slot).wait()
        pe_copy(i, slot).wait()

        def row(r, carry2):
            for j in range(D // 16):
                pe_v = peb[slot, r, pl.ds(j * 16, 16)]
                for b in range(B):
                    xb[slot, r, b, pl.ds(j * 16, 16)] = (
                        xb[slot, r, b, pl.ds(j * 16, 16)] + pe_v)
            return carry2

        lax.fori_loop(0, R, row, 0)
        o_copy(i, slot).start()
        return carry

    lax.fori_loop(0, NCH, step, 0)

    for k in range(NB):
        i = NCH - NB + k
        o_copy(i, i % NB).wait()


def kernel(x, pos_emb):
    seq_len, batch, d_model = x.shape

    sc = pl.kernel(
        _sc_body,
        out_type=jax.ShapeDtypeStruct((S_SC, batch, d_model), x.dtype),
        mesh=plsc.VectorSubcoreMesh(core_axis_name="c", subcore_axis_name="s"),
        scratch_types=[
            pltpu.VMEM((NB, R, B, D), x.dtype),
            pltpu.VMEM((NB, R, D), x.dtype),
            pltpu.SemaphoreType.DMA((NB,)),
            pltpu.SemaphoreType.DMA((NB,)),
            pltpu.SemaphoreType.DMA((NB,)),
        ],
    )
    out_sc = sc(x, pos_emb)

    out_tc = pl.pallas_call(
        _tc_body,
        grid=(S_TC // S_BLK,),
        in_specs=[
            pl.BlockSpec((S_BLK, batch, d_model), lambda i: (i, 0, 0)),
            pl.BlockSpec((S_BLK, d_model), lambda i: (i, 0)),
        ],
        out_specs=pl.BlockSpec((S_BLK, batch, d_model), lambda i: (i, 0, 0)),
        out_shape=jax.ShapeDtypeStruct((seq_len, batch, d_model), x.dtype),
        compiler_params=pltpu.CompilerParams(
            dimension_semantics=("arbitrary",),
        ),
    )(x, pos_emb)

    return lax.dynamic_update_slice(out_tc, out_sc, (S_TC, 0, 0))


# speedup vs baseline: 1.8033x; 1.4842x over previous
import jax
import jax.numpy as jnp
from jax.experimental import pallas as pl
from jax.experimental.pallas import tpu as pltpu

CH = 128
NBUF = 8


def _body(x_hbm, pe_hbm, o_hbm, xb, peb, ob, rx, rp, ws):
    n_chunks = x_hbm.shape[0] // CH

    def x_copy(i, slot):
        return pltpu.make_async_copy(
            x_hbm.at[pl.ds(i * CH, CH)], xb.at[slot], rx.at[slot])

    def pe_copy(i, slot):
        return pltpu.make_async_copy(
            pe_hbm.at[pl.ds(i * CH, CH)], peb.at[slot], rp.at[slot])

    def o_copy(i, slot):
        return pltpu.make_async_copy(
            ob.at[slot], o_hbm.at[pl.ds(i * CH, CH)], ws.at[slot])

    for i in range(NBUF - 1):
        x_copy(i, i).start()
        pe_copy(i, i).start()

    def step(i, carry):
        slot = jax.lax.rem(i, NBUF)
        nxt = i + NBUF - 1
        nslot = jax.lax.rem(nxt, NBUF)

        @pl.when(nxt < n_chunks)
        def _():
            x_copy(nxt, nslot).start()
            pe_copy(nxt, nslot).start()

        x_copy(i, slot).wait()
        pe_copy(i, slot).wait()

        @pl.when(i >= NBUF)
        def _():
            o_copy(i - NBUF, slot).wait()

        pe = peb.at[slot][...]
        ob.at[slot][...] = xb.at[slot][...] + pe[:, None, :]
        o_copy(i, slot).start()
        return carry

    jax.lax.fori_loop(0, n_chunks, step, 0)

    for k in range(NBUF):
        i = n_chunks - NBUF + k
        o_copy(i, i % NBUF).wait()


def kernel(x, pos_emb):
    seq_len, batch, d_model = x.shape
    return pl.pallas_call(
        _body,
        in_specs=[
            pl.BlockSpec(memory_space=pl.ANY),
            pl.BlockSpec(memory_space=pl.ANY),
        ],
        out_specs=pl.BlockSpec(memory_space=pl.ANY),
        out_shape=jax.ShapeDtypeStruct((seq_len, batch, d_model), x.dtype),
        scratch_shapes=[
            pltpu.VMEM((NBUF, CH, batch, d_model), x.dtype),
            pltpu.VMEM((NBUF, CH, d_model), x.dtype),
            pltpu.VMEM((NBUF, CH, batch, d_model), x.dtype),
            pltpu.SemaphoreType.DMA((NBUF,)),
            pltpu.SemaphoreType.DMA((NBUF,)),
            pltpu.SemaphoreType.DMA((NBUF,)),
        ],
    )(x, pos_emb)
